# baseline (speedup 1.0000x reference)
import functools

import jax
import jax.numpy as jnp
from jax import lax
from jax.experimental import pallas as pl
from jax.experimental.pallas import tpu as pltpu
from jax.experimental.pallas import tpu_sc as plsc

_N = 10000
_E = 320000
_IN_F = 128
_HID = 16
_OUT_F = 64
_GAMMA = 1.0

_NC = 2
_NS = 16
_NW = _NC * _NS
_EPW = _E // _NW
_K = 80
_NCHUNK = _EPW // _K
_NBUF = 4
_K2 = 40
_SLAB = _N // _NS
_ZSLAB = 640
_NPAD = _NS * _ZSLAB

_RB = 400
_GRID = _N // _RB

_DEGW = 16


def _degree_partials(dst3):
    mesh = plsc.VectorSubcoreMesh(core_axis_name="c", subcore_axis_name="s")

    @functools.partial(
        pl.kernel,
        mesh=mesh,
        out_type=jax.ShapeDtypeStruct((_NC, _NPAD, _DEGW), jnp.float32),
        scratch_types=[
            pltpu.VMEM((_NCHUNK, _K), jnp.int32),
            pltpu.VMEM((_K, _DEGW), jnp.float32),
            pltpu.VMEM((_K, _DEGW), jnp.float32),
            pltpu.VMEM_SHARED((_NPAD, _DEGW), jnp.float32),
            pltpu.SemaphoreType.DMA,
            pltpu.SemaphoreType.DMA,
        ],
        compiler_params=pltpu.CompilerParams(use_tc_tiling_on_sc=False),
    )
    def deg_kernel(dst_hbm, out, dst_v, ones_v, zero_v, acc, isem, ssem):
        cid = lax.axis_index("c")
        sid = lax.axis_index("s")
        wid = cid * _NS + sid

        idx_load = pltpu.async_copy(dst_hbm.at[wid], dst_v, isem)

        ones_row = jnp.full((16,), 1.0, jnp.float32)
        zero_row = jnp.zeros((16,), jnp.float32)

        def fill(i, _):
            ones_v[i, :] = ones_row
            zero_v[i, :] = zero_row
            return 0

        lax.fori_loop(0, _K, fill, 0)

        zbase = sid * _ZSLAB
        for j in range(_ZSLAB // _K):
            pltpu.sync_copy(zero_v, acc.at[pl.ds(zbase + j * _K, _K)])
        idx_load.wait()
        plsc.subcore_barrier()

        def fire(i, _):
            pltpu.async_copy(ones_v, acc.at[dst_v.at[i]], ssem, add=True)
            return 0

        lax.fori_loop(0, _NCHUNK, fire, 0)

        def drain(i, _):
            pltpu.make_async_copy(ones_v, acc.at[pl.ds(0, _K)], ssem).wait()
            return 0

        lax.fori_loop(0, _NCHUNK, drain, 0)
        plsc.subcore_barrier()

        rbase = sid * _ZSLAB
        pltpu.sync_copy(acc.at[pl.ds(rbase, _ZSLAB)],
                        out.at[cid, pl.ds(rbase, _ZSLAB)])

    return deg_kernel(dst3)


def _segment_sum_partials(table, src3, dst3, feat, k):
    nchunk = _EPW // k
    mesh = plsc.VectorSubcoreMesh(core_axis_name="c", subcore_axis_name="s")

    @functools.partial(
        pl.kernel,
        mesh=mesh,
        out_type=jax.ShapeDtypeStruct((_NC, _NPAD, feat), jnp.float32),
        scratch_types=[
            pltpu.VMEM((nchunk, k), jnp.int32),
            pltpu.VMEM((nchunk, k), jnp.int32),
            pltpu.VMEM((_NBUF, k, feat), jnp.float32),
            pltpu.SemaphoreType.DMA,
            pltpu.SemaphoreType.DMA((_NBUF,)),
            pltpu.SemaphoreType.DMA((_NBUF,)),
            pltpu.VMEM_SHARED((_NPAD, feat), jnp.float32),
        ],
        compiler_params=pltpu.CompilerParams(use_tc_tiling_on_sc=False),
    )
    def seg_kernel(tab, src_hbm, dst_hbm, out, src_v, dst_v, rows, isem,
                   gsem, ssem, acc):
        cid = lax.axis_index("c")
        sid = lax.axis_index("s")
        wid = cid * _NS + sid

        il0 = pltpu.async_copy(src_hbm.at[wid], src_v, isem)
        il1 = pltpu.async_copy(dst_hbm.at[wid], dst_v, isem)

        zero_row = jnp.zeros((16,), jnp.float32)

        def fill(i, _):
            for j in range(feat // 16):
                rows[0, i, pl.ds(j * 16, 16)] = zero_row
            return 0

        lax.fori_loop(0, k, fill, 0)

        zbase = sid * _ZSLAB
        for j in range(_ZSLAB // k):
            pltpu.sync_copy(rows.at[0], acc.at[pl.ds(zbase + j * k, k)])
        il0.wait()
        il1.wait()
        plsc.subcore_barrier()

        def gather(c, b):
            pltpu.async_copy(tab.at[src_v.at[c]], rows.at[b], gsem.at[b])

        def gwait(b):
            pltpu.make_async_copy(tab.at[src_v.at[0]], rows.at[b],
                                  gsem.at[b]).wait()

        def scatter(c, b):
            pltpu.async_copy(rows.at[b], acc.at[dst_v.at[c]], ssem.at[b],
                             add=True)

        def swait(b):
            pltpu.make_async_copy(rows.at[b], acc.at[pl.ds(0, k)],
                                  ssem.at[b]).wait()

        gather(0, 0)
        gather(1, 1)
        for c in range(2):
            gwait(c)
            scatter(c, c)
            gather(c + 2, c + 2)

        def step(c, _):
            b = lax.rem(c, _NBUF)
            bnn = lax.rem(c + 2, _NBUF)
            gwait(b)
            scatter(c, b)
            swait(bnn)
            gather(c + 2, bnn)
            return 0

        lax.fori_loop(2, nchunk - 2, step, 0)
        for c in range(nchunk - 2, nchunk):
            b = c % _NBUF
            gwait(b)
            scatter(c, b)
        for b in range(_NBUF):
            swait(b)
        plsc.subcore_barrier()

        rbase = sid * _ZSLAB
        pltpu.sync_copy(acc.at[pl.ds(rbase, _ZSLAB)],
                        out.at[cid, pl.ds(rbase, _ZSLAB)])

    return seg_kernel(table, src3, dst3)


def _norms(d_ref):
    deg = jnp.maximum(d_ref[0][:, 0:1] + d_ref[1][:, 0:1], 1.0)
    n1 = lax.rsqrt(deg)
    n2 = 1.0 / deg
    return n1, n2


def _dense1_body(x_ref, w_ref, b_ref, d_ref, o_ref):
    n1, n2 = _norms(d_ref)
    h = jnp.dot(x_ref[...], w_ref[...], preferred_element_type=jnp.float32)
    h = jnp.maximum(h + b_ref[...], 0.0)
    hm = h[:, :_HID]
    hv = h[:, _HID:]
    att = jnp.exp(-_GAMMA * hv)
    o_ref[...] = jnp.concatenate([hm * att * n1, hv * att * att * n2], axis=1)


def _dense2_body(s_ref, wm_ref, wv_ref, bm_ref, bv_ref, d_ref, o_ref):
    n1, n2 = _norms(d_ref)
    s = s_ref[0] + s_ref[1]
    mean_in = s[:, :_HID] * n1
    var_in = s[:, _HID:] * n2
    hm = jnp.dot(mean_in, wm_ref[...],
                 preferred_element_type=jnp.float32) + bm_ref[...]
    hv = jnp.dot(var_in, wv_ref[...],
                 preferred_element_type=jnp.float32) + bv_ref[...]
    hv = jnp.maximum(hv, 0.0)
    att = jnp.exp(-_GAMMA * hv)
    o_ref[...] = jnp.concatenate([hm * att * n1, hv * att * att * n2], axis=1)


def _final_body(s_ref, d_ref, e_ref, o_ref):
    n1, n2 = _norms(d_ref)
    s = s_ref[0] + s_ref[1]
    mean = s[:, :_OUT_F] * n1
    var = s[:, _OUT_F:] * n2
    o_ref[...] = e_ref[...] * jnp.sqrt(var + 1e-8) + mean


def _dense1(x, w1, b1, degp):
    return pl.pallas_call(
        _dense1_body,
        grid=(_GRID,),
        in_specs=[
            pl.BlockSpec((_RB, _IN_F), lambda i: (i, 0)),
            pl.BlockSpec((_IN_F, 2 * _HID), lambda i: (0, 0)),
            pl.BlockSpec((1, 2 * _HID), lambda i: (0, 0)),
            pl.BlockSpec((2, _RB, _DEGW), lambda i: (0, i, 0)),
        ],
        out_specs=pl.BlockSpec((_RB, 2 * _HID), lambda i: (i, 0)),
        out_shape=jax.ShapeDtypeStruct((_N, 2 * _HID), jnp.float32),
    )(x, w1, b1, degp)


def _dense2(s1, wm2, wv2, bm2, bv2, degp):
    return pl.pallas_call(
        _dense2_body,
        grid=(_GRID,),
        in_specs=[
            pl.BlockSpec((2, _RB, 2 * _HID), lambda i: (0, i, 0)),
            pl.BlockSpec((_HID, _OUT_F), lambda i: (0, 0)),
            pl.BlockSpec((_HID, _OUT_F), lambda i: (0, 0)),
            pl.BlockSpec((1, _OUT_F), lambda i: (0, 0)),
            pl.BlockSpec((1, _OUT_F), lambda i: (0, 0)),
            pl.BlockSpec((2, _RB, _DEGW), lambda i: (0, i, 0)),
        ],
        out_specs=pl.BlockSpec((_RB, 2 * _OUT_F), lambda i: (i, 0)),
        out_shape=jax.ShapeDtypeStruct((_N, 2 * _OUT_F), jnp.float32),
    )(s1, wm2, wv2, bm2, bv2, degp)


def _final(s2, degp, eps):
    return pl.pallas_call(
        _final_body,
        grid=(_GRID,),
        in_specs=[
            pl.BlockSpec((2, _RB, 2 * _OUT_F), lambda i: (0, i, 0)),
            pl.BlockSpec((2, _RB, _DEGW), lambda i: (0, i, 0)),
            pl.BlockSpec((_RB, _OUT_F), lambda i: (i, 0)),
        ],
        out_specs=pl.BlockSpec((_RB, _OUT_F), lambda i: (i, 0)),
        out_shape=jax.ShapeDtypeStruct((_N, _OUT_F), jnp.float32),
    )(s2, degp, eps)


def kernel(x, edge_index, w_mean1, b_mean1, w_var1, b_var1,
           w_mean2, b_mean2, w_var2, b_var2):
    w1 = jnp.concatenate([w_mean1, w_var1], axis=1)
    b1 = jnp.concatenate([b_mean1, b_var1]).reshape(1, 2 * _HID)
    bm2 = b_mean2.reshape(1, _OUT_F)
    bv2 = b_var2.reshape(1, _OUT_F)

    src3 = edge_index[0].reshape(_NW, _NCHUNK, _K)
    dst3 = edge_index[1].reshape(_NW, _NCHUNK, _K)
    src3n = edge_index[0].reshape(_NW, _EPW // _K2, _K2)
    dst3n = edge_index[1].reshape(_NW, _EPW // _K2, _K2)
    degp = _degree_partials(dst3)
    t1 = _dense1(x, w1, b1, degp)
    s1 = _segment_sum_partials(t1, src3, dst3, 2 * _HID, _K)
    t2 = _dense2(s1, w_mean2, w_var2, bm2, bv2, degp)
    s2 = _segment_sum_partials(t2, src3n, dst3n, 2 * _OUT_F, _K2)
    eps = jax.random.normal(jax.random.key(42), (_N, _OUT_F), jnp.float32)
    return _final(s2, degp, eps)


# speedup vs baseline: 1.0405x; 1.0405x over previous
import functools

import jax
import jax.numpy as jnp
from jax import lax
from jax.experimental import pallas as pl
from jax.experimental.pallas import tpu as pltpu
from jax.experimental.pallas import tpu_sc as plsc

_N = 10000
_E = 320000
_IN_F = 128
_HID = 16
_OUT_F = 64
_GAMMA = 1.0

_NC = 2
_NS = 16
_NW = _NC * _NS
_EPW = _E // _NW
_K = 80
_NCHUNK = _EPW // _K
_NBUF = 4
_K2 = 40
_SLAB = _N // _NS
_ZSLAB = 640
_NPAD = _NS * _ZSLAB

_RB = 400
_GRID = _N // _RB

_DEGW = 16


def _degree_partials(dst3):
    mesh = plsc.VectorSubcoreMesh(core_axis_name="c", subcore_axis_name="s")

    @functools.partial(
        pl.kernel,
        mesh=mesh,
        out_type=jax.ShapeDtypeStruct((_NC, _NPAD, _DEGW), jnp.float32),
        scratch_types=[
            pltpu.VMEM((_NCHUNK, _K), jnp.int32),
            pltpu.VMEM((_K, _DEGW), jnp.float32),
            pltpu.VMEM((_K, _DEGW), jnp.float32),
            pltpu.VMEM_SHARED((_NPAD, _DEGW), jnp.float32),
            pltpu.SemaphoreType.DMA,
            pltpu.SemaphoreType.DMA,
        ],
        compiler_params=pltpu.CompilerParams(use_tc_tiling_on_sc=False),
    )
    def deg_kernel(dst_hbm, out, dst_v, ones_v, zero_v, acc, isem, ssem):
        cid = lax.axis_index("c")
        sid = lax.axis_index("s")
        wid = cid * _NS + sid

        idx_load = pltpu.async_copy(dst_hbm.at[wid], dst_v, isem)

        ones_row = jnp.full((16,), 1.0, jnp.float32)
        zero_row = jnp.zeros((16,), jnp.float32)

        def fill(i, _):
            ones_v[i, :] = ones_row
            zero_v[i, :] = zero_row
            return 0

        lax.fori_loop(0, _K, fill, 0)

        zbase = sid * _ZSLAB
        for j in range(_ZSLAB // _K):
            pltpu.sync_copy(zero_v, acc.at[pl.ds(zbase + j * _K, _K)])
        idx_load.wait()
        plsc.subcore_barrier()

        def fire(i, _):
            pltpu.async_copy(ones_v, acc.at[dst_v.at[i]], ssem, add=True)
            return 0

        lax.fori_loop(0, _NCHUNK, fire, 0)

        def drain(i, _):
            pltpu.make_async_copy(ones_v, acc.at[pl.ds(0, _K)], ssem).wait()
            return 0

        lax.fori_loop(0, _NCHUNK, drain, 0)
        plsc.subcore_barrier()

        rbase = sid * _ZSLAB
        pltpu.sync_copy(acc.at[pl.ds(rbase, _ZSLAB)],
                        out.at[cid, pl.ds(rbase, _ZSLAB)])

    return deg_kernel(dst3)


def _segment_sum_partials(table, src3, dst3, feat, k):
    nchunk = _EPW // k
    mesh = plsc.VectorSubcoreMesh(core_axis_name="c", subcore_axis_name="s")

    @functools.partial(
        pl.kernel,
        mesh=mesh,
        out_type=jax.ShapeDtypeStruct((_NC, _NPAD, feat), jnp.float32),
        scratch_types=[
            pltpu.VMEM((nchunk, k), jnp.int32),
            pltpu.VMEM((nchunk, k), jnp.int32),
            pltpu.VMEM((_NBUF, k, feat), jnp.float32),
            pltpu.SemaphoreType.DMA,
            pltpu.SemaphoreType.DMA((_NBUF,)),
            pltpu.SemaphoreType.DMA((_NBUF,)),
            pltpu.VMEM_SHARED((_NPAD, feat), jnp.float32),
        ],
        compiler_params=pltpu.CompilerParams(use_tc_tiling_on_sc=False),
    )
    def seg_kernel(tab, src_hbm, dst_hbm, out, src_v, dst_v, rows, isem,
                   gsem, ssem, acc):
        cid = lax.axis_index("c")
        sid = lax.axis_index("s")
        wid = cid * _NS + sid

        il0 = pltpu.async_copy(src_hbm.at[wid], src_v, isem)
        il1 = pltpu.async_copy(dst_hbm.at[wid], dst_v, isem)

        zero_row = jnp.zeros((16,), jnp.float32)

        def fill(i, _):
            for j in range(feat // 16):
                rows[0, i, pl.ds(j * 16, 16)] = zero_row
            return 0

        lax.fori_loop(0, k, fill, 0)

        zbase = sid * _ZSLAB
        for j in range(_ZSLAB // k):
            pltpu.sync_copy(rows.at[0], acc.at[pl.ds(zbase + j * k, k)])
        il0.wait()
        il1.wait()
        plsc.subcore_barrier()

        def gather(c, b):
            pltpu.async_copy(tab.at[src_v.at[c]], rows.at[b], gsem.at[b])

        def gwait(b):
            pltpu.make_async_copy(tab.at[src_v.at[0]], rows.at[b],
                                  gsem.at[b]).wait()

        def scatter(c, b):
            pltpu.async_copy(rows.at[b], acc.at[dst_v.at[c]], ssem.at[b],
                             add=True)

        def swait(b):
            pltpu.make_async_copy(rows.at[b], acc.at[pl.ds(0, k)],
                                  ssem.at[b]).wait()

        gather(0, 0)
        gather(1, 1)
        for c in range(2):
            gwait(c)
            scatter(c, c)
            gather(c + 2, c + 2)

        def step(c, _):
            b = lax.rem(c, _NBUF)
            bnn = lax.rem(c + 2, _NBUF)
            gwait(b)
            scatter(c, b)
            swait(bnn)
            gather(c + 2, bnn)
            return 0

        lax.fori_loop(2, nchunk - 2, step, 0)
        for c in range(nchunk - 2, nchunk):
            b = c % _NBUF
            gwait(b)
            scatter(c, b)
        for b in range(_NBUF):
            swait(b)
        plsc.subcore_barrier()

        rbase = sid * _ZSLAB
        pltpu.sync_copy(acc.at[pl.ds(rbase, _ZSLAB)],
                        out.at[cid, pl.ds(rbase, _ZSLAB)])

    return seg_kernel(table, src3, dst3)


def _segment_sum_partials_sync(table, src3, dst3, feat):
    mesh = plsc.VectorSubcoreMesh(core_axis_name="c", subcore_axis_name="s")

    @functools.partial(
        pl.kernel,
        mesh=mesh,
        out_type=jax.ShapeDtypeStruct((_NC, _NPAD, feat), jnp.float32),
        scratch_types=[
            pltpu.VMEM((_NCHUNK, _K), jnp.int32),
            pltpu.VMEM((_NCHUNK, _K), jnp.int32),
            pltpu.VMEM((_K, feat), jnp.float32),
            pltpu.VMEM((_K, feat), jnp.float32),
            pltpu.SemaphoreType.DMA,
            pltpu.SemaphoreType.DMA,
            pltpu.SemaphoreType.DMA,
            pltpu.VMEM_SHARED((_NPAD, feat), jnp.float32),
        ],
        compiler_params=pltpu.CompilerParams(use_tc_tiling_on_sc=False),
    )
    def seg_kernel(tab, src_hbm, dst_hbm, out, src_v, dst_v, rows_a, rows_b,
                   isem, sem_a, sem_b, acc):
        cid = lax.axis_index("c")
        sid = lax.axis_index("s")
        wid = cid * _NS + sid

        il0 = pltpu.async_copy(src_hbm.at[wid], src_v, isem)
        il1 = pltpu.async_copy(dst_hbm.at[wid], dst_v, isem)

        zero_row = jnp.zeros((16,), jnp.float32)

        def fill(i, _):
            for j in range(feat // 16):
                rows_a[i, pl.ds(j * 16, 16)] = zero_row
            return 0

        lax.fori_loop(0, _K, fill, 0)

        zbase = sid * _ZSLAB
        for j in range(_ZSLAB // _K):
            pltpu.sync_copy(rows_a, acc.at[pl.ds(zbase + j * _K, _K)])
        il0.wait()
        il1.wait()
        plsc.subcore_barrier()

        def gather(c, buf, sem):
            pltpu.async_copy(tab.at[src_v.at[c]], buf, sem)

        def gwait(buf, sem):
            pltpu.make_async_copy(tab.at[src_v.at[0]], buf, sem).wait()

        def scatter(c, buf):
            pltpu.sync_copy(buf, acc.at[dst_v.at[c]], add=True)

        gather(0, rows_a, sem_a)

        def pair(i, _):
            c = 2 * i
            gather(c + 1, rows_b, sem_b)
            gwait(rows_a, sem_a)
            scatter(c, rows_a)
            gather(c + 2, rows_a, sem_a)
            gwait(rows_b, sem_b)
            scatter(c + 1, rows_b)
            return 0

        lax.fori_loop(0, (_NCHUNK - 1) // 2, pair, 0)
        gwait(rows_a, sem_a)
        scatter(_NCHUNK - 1, rows_a)
        plsc.subcore_barrier()

        rbase = sid * _ZSLAB
        pltpu.sync_copy(acc.at[pl.ds(rbase, _ZSLAB)],
                        out.at[cid, pl.ds(rbase, _ZSLAB)])

    return seg_kernel(table, src3, dst3)


def _norms(d_ref):
    deg = jnp.maximum(d_ref[0][:, 0:1] + d_ref[1][:, 0:1], 1.0)
    n1 = lax.rsqrt(deg)
    n2 = 1.0 / deg
    return n1, n2


def _dense1_body(x_ref, w_ref, b_ref, o_ref):
    h = jnp.dot(x_ref[...], w_ref[...], preferred_element_type=jnp.float32)
    h = jnp.maximum(h + b_ref[...], 0.0)
    hm = h[:, :_HID]
    hv = h[:, _HID:]
    att = jnp.exp(-_GAMMA * hv)
    o_ref[...] = jnp.concatenate([hm * att, hv * att * att], axis=1)


def _scale1_body(h_ref, d_ref, o_ref):
    n1, n2 = _norms(d_ref)
    o_ref[...] = jnp.concatenate([h_ref[:, :_HID] * n1, h_ref[:, _HID:] * n2],
                                 axis=1)


def _dense2_body(s_ref, wm_ref, wv_ref, bm_ref, bv_ref, d_ref, o_ref):
    n1, n2 = _norms(d_ref)
    s = s_ref[0] + s_ref[1]
    mean_in = s[:, :_HID] * n1
    var_in = s[:, _HID:] * n2
    hm = jnp.dot(mean_in, wm_ref[...],
                 preferred_element_type=jnp.float32) + bm_ref[...]
    hv = jnp.dot(var_in, wv_ref[...],
                 preferred_element_type=jnp.float32) + bv_ref[...]
    hv = jnp.maximum(hv, 0.0)
    att = jnp.exp(-_GAMMA * hv)
    o_ref[...] = jnp.concatenate([hm * att * n1, hv * att * att * n2], axis=1)


def _final_body(s_ref, d_ref, e_ref, o_ref):
    n1, n2 = _norms(d_ref)
    s = s_ref[0] + s_ref[1]
    mean = s[:, :_OUT_F] * n1
    var = s[:, _OUT_F:] * n2
    o_ref[...] = e_ref[...] * jnp.sqrt(var + 1e-8) + mean


def _dense1(x, w1, b1):
    return pl.pallas_call(
        _dense1_body,
        grid=(_GRID,),
        in_specs=[
            pl.BlockSpec((_RB, _IN_F), lambda i: (i, 0)),
            pl.BlockSpec((_IN_F, 2 * _HID), lambda i: (0, 0)),
            pl.BlockSpec((1, 2 * _HID), lambda i: (0, 0)),
        ],
        out_specs=pl.BlockSpec((_RB, 2 * _HID), lambda i: (i, 0)),
        out_shape=jax.ShapeDtypeStruct((_N, 2 * _HID), jnp.float32),
    )(x, w1, b1)


def _scale1(h, degp):
    return pl.pallas_call(
        _scale1_body,
        grid=(_GRID,),
        in_specs=[
            pl.BlockSpec((_RB, 2 * _HID), lambda i: (i, 0)),
            pl.BlockSpec((2, _RB, _DEGW), lambda i: (0, i, 0)),
        ],
        out_specs=pl.BlockSpec((_RB, 2 * _HID), lambda i: (i, 0)),
        out_shape=jax.ShapeDtypeStruct((_N, 2 * _HID), jnp.float32),
    )(h, degp)


def _dense2(s1, wm2, wv2, bm2, bv2, degp):
    return pl.pallas_call(
        _dense2_body,
        grid=(_GRID,),
        in_specs=[
            pl.BlockSpec((2, _RB, 2 * _HID), lambda i: (0, i, 0)),
            pl.BlockSpec((_HID, _OUT_F), lambda i: (0, 0)),
            pl.BlockSpec((_HID, _OUT_F), lambda i: (0, 0)),
            pl.BlockSpec((1, _OUT_F), lambda i: (0, 0)),
            pl.BlockSpec((1, _OUT_F), lambda i: (0, 0)),
            pl.BlockSpec((2, _RB, _DEGW), lambda i: (0, i, 0)),
        ],
        out_specs=pl.BlockSpec((_RB, 2 * _OUT_F), lambda i: (i, 0)),
        out_shape=jax.ShapeDtypeStruct((_N, 2 * _OUT_F), jnp.float32),
    )(s1, wm2, wv2, bm2, bv2, degp)


def _final(s2, degp, eps):
    return pl.pallas_call(
        _final_body,
        grid=(_GRID,),
        in_specs=[
            pl.BlockSpec((2, _RB, 2 * _OUT_F), lambda i: (0, i, 0)),
            pl.BlockSpec((2, _RB, _DEGW), lambda i: (0, i, 0)),
            pl.BlockSpec((_RB, _OUT_F), lambda i: (i, 0)),
        ],
        out_specs=pl.BlockSpec((_RB, _OUT_F), lambda i: (i, 0)),
        out_shape=jax.ShapeDtypeStruct((_N, _OUT_F), jnp.float32),
    )(s2, degp, eps)


def kernel(x, edge_index, w_mean1, b_mean1, w_var1, b_var1,
           w_mean2, b_mean2, w_var2, b_var2):
    w1 = jnp.concatenate([w_mean1, w_var1], axis=1)
    b1 = jnp.concatenate([b_mean1, b_var1]).reshape(1, 2 * _HID)
    bm2 = b_mean2.reshape(1, _OUT_F)
    bv2 = b_var2.reshape(1, _OUT_F)

    src3 = edge_index[0].reshape(_NW, _NCHUNK, _K)
    dst3 = edge_index[1].reshape(_NW, _NCHUNK, _K)
    src3n = edge_index[0].reshape(_NW, _EPW // _K2, _K2)
    dst3n = edge_index[1].reshape(_NW, _EPW // _K2, _K2)
    degp = _degree_partials(dst3)
    h1 = _dense1(x, w1, b1)
    t1 = _scale1(h1, degp)
    s1 = _segment_sum_partials(t1, src3, dst3, 2 * _HID, _K)
    t2 = _dense2(s1, w_mean2, w_var2, bm2, bv2, degp)
    s2 = _segment_sum_partials_sync(t2, src3, dst3, 2 * _OUT_F)
    eps = jax.random.normal(jax.random.key(42), (_N, _OUT_F), jnp.float32)
    return _final(s2, degp, eps)


# speedup vs baseline: 1.1035x; 1.0606x over previous
import functools

import jax
import jax.numpy as jnp
from jax import lax
from jax.experimental import pallas as pl
from jax.experimental.pallas import tpu as pltpu
from jax.experimental.pallas import tpu_sc as plsc

_N = 10000
_E = 320000
_IN_F = 128
_HID = 16
_OUT_F = 64
_GAMMA = 1.0

_NC = 2
_NS = 16
_NW = _NC * _NS
_EPW = _E // _NW
_K = 80
_NCHUNK = _EPW // _K
_NBUF = 4
_SLAB = _N // _NS
_ZSLAB = 640
_NPAD = _NS * _ZSLAB

_RB = 400
_GRID = _N // _RB

_DEGW = 16


def _degree_partials(dst3):
    mesh = plsc.VectorSubcoreMesh(core_axis_name="c", subcore_axis_name="s")

    @functools.partial(
        pl.kernel,
        mesh=mesh,
        out_type=jax.ShapeDtypeStruct((_NC, _NPAD, _DEGW), jnp.float32),
        scratch_types=[
            pltpu.VMEM((_NCHUNK, _K), jnp.int32),
            pltpu.VMEM((_K, _DEGW), jnp.float32),
            pltpu.VMEM((_K, _DEGW), jnp.float32),
            pltpu.VMEM_SHARED((_NPAD, _DEGW), jnp.float32),
            pltpu.SemaphoreType.DMA,
            pltpu.SemaphoreType.DMA,
        ],
        compiler_params=pltpu.CompilerParams(use_tc_tiling_on_sc=False),
    )
    def deg_kernel(dst_hbm, out, dst_v, ones_v, zero_v, acc, isem, ssem):
        cid = lax.axis_index("c")
        sid = lax.axis_index("s")
        wid = cid * _NS + sid

        idx_load = pltpu.async_copy(dst_hbm.at[wid], dst_v, isem)

        ones_row = jnp.full((16,), 1.0, jnp.float32)
        zero_row = jnp.zeros((16,), jnp.float32)

        def fill(i, _):
            ones_v[i, :] = ones_row
            zero_v[i, :] = zero_row
            return 0

        lax.fori_loop(0, _K, fill, 0)

        zbase = sid * _ZSLAB
        for j in range(_ZSLAB // _K):
            pltpu.sync_copy(zero_v, acc.at[pl.ds(zbase + j * _K, _K)])
        idx_load.wait()
        plsc.subcore_barrier()

        def fire(i, _):
            pltpu.async_copy(ones_v, acc.at[dst_v.at[i]], ssem, add=True)
            return 0

        lax.fori_loop(0, _NCHUNK, fire, 0)

        def drain(i, _):
            pltpu.make_async_copy(ones_v, acc.at[pl.ds(0, _K)], ssem).wait()
            return 0

        lax.fori_loop(0, _NCHUNK, drain, 0)
        plsc.subcore_barrier()

        rbase = sid * _ZSLAB
        pltpu.sync_copy(acc.at[pl.ds(rbase, _ZSLAB)],
                        out.at[cid, pl.ds(rbase, _ZSLAB)])

    return deg_kernel(dst3)


def _segment_sum_partials(table, src3, dst3, feat, k):
    nchunk = _EPW // k
    mesh = plsc.VectorSubcoreMesh(core_axis_name="c", subcore_axis_name="s")

    @functools.partial(
        pl.kernel,
        mesh=mesh,
        out_type=jax.ShapeDtypeStruct((_NC, _NPAD, feat), jnp.float32),
        scratch_types=[
            pltpu.VMEM((nchunk, k), jnp.int32),
            pltpu.VMEM((nchunk, k), jnp.int32),
            pltpu.VMEM((_NBUF, k, feat), jnp.float32),
            pltpu.SemaphoreType.DMA,
            pltpu.SemaphoreType.DMA((_NBUF,)),
            pltpu.SemaphoreType.DMA((_NBUF,)),
            pltpu.VMEM_SHARED((_NPAD, feat), jnp.float32),
        ],
        compiler_params=pltpu.CompilerParams(use_tc_tiling_on_sc=False),
    )
    def seg_kernel(tab, src_hbm, dst_hbm, out, src_v, dst_v, rows, isem,
                   gsem, ssem, acc):
        cid = lax.axis_index("c")
        sid = lax.axis_index("s")
        wid = cid * _NS + sid

        il0 = pltpu.async_copy(src_hbm.at[wid], src_v, isem)
        il1 = pltpu.async_copy(dst_hbm.at[wid], dst_v, isem)

        zero_row = jnp.zeros((16,), jnp.float32)

        def fill(i, _):
            for j in range(feat // 16):
                rows[0, i, pl.ds(j * 16, 16)] = zero_row
            return 0

        lax.fori_loop(0, k, fill, 0)

        zbase = sid * _ZSLAB
        for j in range(_ZSLAB // k):
            pltpu.sync_copy(rows.at[0], acc.at[pl.ds(zbase + j * k, k)])
        il0.wait()
        il1.wait()
        plsc.subcore_barrier()

        def gather(c, b):
            pltpu.async_copy(tab.at[src_v.at[c]], rows.at[b], gsem.at[b])

        def gwait(b):
            pltpu.make_async_copy(tab.at[src_v.at[0]], rows.at[b],
                                  gsem.at[b]).wait()

        def scatter(c, b):
            pltpu.async_copy(rows.at[b], acc.at[dst_v.at[c]], ssem.at[b],
                             add=True)

        def swait(b):
            pltpu.make_async_copy(rows.at[b], acc.at[pl.ds(0, k)],
                                  ssem.at[b]).wait()

        gather(0, 0)
        gather(1, 1)
        for c in range(2):
            gwait(c)
            scatter(c, c)
            gather(c + 2, c + 2)

        def step(c, _):
            b = lax.rem(c, _NBUF)
            bnn = lax.rem(c + 2, _NBUF)
            gwait(b)
            scatter(c, b)
            swait(bnn)
            gather(c + 2, bnn)
            return 0

        lax.fori_loop(2, nchunk - 2, step, 0)
        for c in range(nchunk - 2, nchunk):
            b = c % _NBUF
            gwait(b)
            scatter(c, b)
        for b in range(_NBUF):
            swait(b)
        plsc.subcore_barrier()

        rbase = sid * _ZSLAB
        pltpu.sync_copy(acc.at[pl.ds(rbase, _ZSLAB)],
                        out.at[cid, pl.ds(rbase, _ZSLAB)])

    return seg_kernel(table, src3, dst3)


def _segment_sum_partials_big(table, src3, dst3, feat):
    _NB = 3
    _ND = 8
    mesh = plsc.VectorSubcoreMesh(core_axis_name="c", subcore_axis_name="s")

    @functools.partial(
        pl.kernel,
        mesh=mesh,
        out_type=jax.ShapeDtypeStruct((_NC, _NPAD, feat), jnp.float32),
        scratch_types=[
            pltpu.VMEM((_NCHUNK, _K), jnp.int32),
            pltpu.VMEM((_ND, _K), jnp.int32),
            pltpu.VMEM((_NB, _K, feat), jnp.float32),
            pltpu.SemaphoreType.DMA,
            pltpu.SemaphoreType.DMA((_ND,)),
            pltpu.SemaphoreType.DMA((_NB,)),
            pltpu.SemaphoreType.DMA((_NB,)),
            pltpu.VMEM_SHARED((_NPAD, feat), jnp.float32),
        ],
        compiler_params=pltpu.CompilerParams(use_tc_tiling_on_sc=False),
    )
    def seg_kernel(tab, src_hbm, dst_hbm, out, src_v, dring, rows, isem,
                   dsem, gsem, ssem, acc):
        cid = lax.axis_index("c")
        sid = lax.axis_index("s")
        wid = cid * _NS + sid

        def dload(c):
            s = lax.rem(c, _ND) if not isinstance(c, int) else c % _ND
            pltpu.async_copy(dst_hbm.at[wid, c], dring.at[s], dsem.at[s])

        def iwait(c):
            s = lax.rem(c, _ND) if not isinstance(c, int) else c % _ND
            pltpu.make_async_copy(dst_hbm.at[wid, 0], dring.at[s],
                                  dsem.at[s]).wait()

        il0 = pltpu.async_copy(src_hbm.at[wid], src_v, isem)
        for c in range(_ND - 2):
            dload(c)

        zero_row = jnp.zeros((16,), jnp.float32)

        def fill(i, _):
            for j in range(feat // 16):
                rows[0, i, pl.ds(j * 16, 16)] = zero_row
            return 0

        lax.fori_loop(0, _K, fill, 0)

        zbase = sid * _ZSLAB
        for j in range(_ZSLAB // _K):
            pltpu.sync_copy(rows.at[0], acc.at[pl.ds(zbase + j * _K, _K)])
        il0.wait()
        plsc.subcore_barrier()

        def gather(c, b):
            pltpu.async_copy(tab.at[src_v.at[c]], rows.at[b], gsem.at[b])

        def gwait(b):
            pltpu.make_async_copy(tab.at[src_v.at[0]], rows.at[b],
                                  gsem.at[b]).wait()

        def scatter(c, b):
            s = lax.rem(c, _ND) if not isinstance(c, int) else c % _ND
            pltpu.async_copy(rows.at[b], acc.at[dring.at[s]], ssem.at[b],
                             add=True)

        def swait(b):
            pltpu.make_async_copy(rows.at[b], acc.at[pl.ds(0, _K)],
                                  ssem.at[b]).wait()

        gather(0, 0)
        gather(1, 1)
        gwait(0)
        iwait(0)
        scatter(0, 0)
        gather(2, 2)
        dload(_ND - 2)
        gwait(1)
        iwait(1)
        scatter(1, 1)
        swait(0)
        gather(3, 0)
        dload(_ND - 1)

        def step(c, _):
            b = lax.rem(c, _NB)
            b2 = lax.rem(c + 2, _NB)
            gwait(b)
            iwait(c)
            scatter(c, b)
            swait(b2)
            gather(c + 2, b2)

            @pl.when(c + _ND - 2 < _NCHUNK)
            def _():
                dload(c + _ND - 2)

            return 0

        lax.fori_loop(2, _NCHUNK - 2, step, 0)
        for c in range(_NCHUNK - 2, _NCHUNK):
            b = c % _NB
            gwait(b)
            iwait(c)
            scatter(c, b)
        for b in range(_NB):
            swait(b)
        plsc.subcore_barrier()

        rbase = sid * _ZSLAB
        pltpu.sync_copy(acc.at[pl.ds(rbase, _ZSLAB)],
                        out.at[cid, pl.ds(rbase, _ZSLAB)])

    return seg_kernel(table, src3, dst3)


def _norms(d_ref):
    deg = jnp.maximum(d_ref[0][:, 0:1] + d_ref[1][:, 0:1], 1.0)
    n1 = lax.rsqrt(deg)
    n2 = 1.0 / deg
    return n1, n2


def _dense1_body(x_ref, w_ref, b_ref, o_ref):
    h = jnp.dot(x_ref[...], w_ref[...], preferred_element_type=jnp.float32)
    h = jnp.maximum(h + b_ref[...], 0.0)
    hm = h[:, :_HID]
    hv = h[:, _HID:]
    att = jnp.exp(-_GAMMA * hv)
    o_ref[...] = jnp.concatenate([hm * att, hv * att * att], axis=1)


def _scale1_body(h_ref, d_ref, o_ref):
    n1, n2 = _norms(d_ref)
    o_ref[...] = jnp.concatenate([h_ref[:, :_HID] * n1, h_ref[:, _HID:] * n2],
                                 axis=1)


def _dense2_body(s_ref, wm_ref, wv_ref, bm_ref, bv_ref, d_ref, o_ref):
    n1, n2 = _norms(d_ref)
    s = s_ref[0] + s_ref[1]
    mean_in = s[:, :_HID] * n1
    var_in = s[:, _HID:] * n2
    hm = jnp.dot(mean_in, wm_ref[...],
                 preferred_element_type=jnp.float32) + bm_ref[...]
    hv = jnp.dot(var_in, wv_ref[...],
                 preferred_element_type=jnp.float32) + bv_ref[...]
    hv = jnp.maximum(hv, 0.0)
    att = jnp.exp(-_GAMMA * hv)
    o_ref[...] = jnp.concatenate([hm * att * n1, hv * att * att * n2], axis=1)


def _final_body(s_ref, d_ref, e_ref, o_ref):
    n1, n2 = _norms(d_ref)
    s = s_ref[0] + s_ref[1]
    mean = s[:, :_OUT_F] * n1
    var = s[:, _OUT_F:] * n2
    o_ref[...] = e_ref[...] * jnp.sqrt(var + 1e-8) + mean


def _dense1(x, w1, b1):
    return pl.pallas_call(
        _dense1_body,
        grid=(_GRID,),
        in_specs=[
            pl.BlockSpec((_RB, _IN_F), lambda i: (i, 0)),
            pl.BlockSpec((_IN_F, 2 * _HID), lambda i: (0, 0)),
            pl.BlockSpec((1, 2 * _HID), lambda i: (0, 0)),
        ],
        out_specs=pl.BlockSpec((_RB, 2 * _HID), lambda i: (i, 0)),
        out_shape=jax.ShapeDtypeStruct((_N, 2 * _HID), jnp.float32),
    )(x, w1, b1)


def _scale1(h, degp):
    return pl.pallas_call(
        _scale1_body,
        grid=(_GRID,),
        in_specs=[
            pl.BlockSpec((_RB, 2 * _HID), lambda i: (i, 0)),
            pl.BlockSpec((2, _RB, _DEGW), lambda i: (0, i, 0)),
        ],
        out_specs=pl.BlockSpec((_RB, 2 * _HID), lambda i: (i, 0)),
        out_shape=jax.ShapeDtypeStruct((_N, 2 * _HID), jnp.float32),
    )(h, degp)


def _dense2(s1, wm2, wv2, bm2, bv2, degp):
    return pl.pallas_call(
        _dense2_body,
        grid=(_GRID,),
        in_specs=[
            pl.BlockSpec((2, _RB, 2 * _HID), lambda i: (0, i, 0)),
            pl.BlockSpec((_HID, _OUT_F), lambda i: (0, 0)),
            pl.BlockSpec((_HID, _OUT_F), lambda i: (0, 0)),
            pl.BlockSpec((1, _OUT_F), lambda i: (0, 0)),
            pl.BlockSpec((1, _OUT_F), lambda i: (0, 0)),
            pl.BlockSpec((2, _RB, _DEGW), lambda i: (0, i, 0)),
        ],
        out_specs=pl.BlockSpec((_RB, 2 * _OUT_F), lambda i: (i, 0)),
        out_shape=jax.ShapeDtypeStruct((_N, 2 * _OUT_F), jnp.float32),
    )(s1, wm2, wv2, bm2, bv2, degp)


def _final(s2, degp, eps):
    return pl.pallas_call(
        _final_body,
        grid=(_GRID,),
        in_specs=[
            pl.BlockSpec((2, _RB, 2 * _OUT_F), lambda i: (0, i, 0)),
            pl.BlockSpec((2, _RB, _DEGW), lambda i: (0, i, 0)),
            pl.BlockSpec((_RB, _OUT_F), lambda i: (i, 0)),
        ],
        out_specs=pl.BlockSpec((_RB, _OUT_F), lambda i: (i, 0)),
        out_shape=jax.ShapeDtypeStruct((_N, _OUT_F), jnp.float32),
    )(s2, degp, eps)


def kernel(x, edge_index, w_mean1, b_mean1, w_var1, b_var1,
           w_mean2, b_mean2, w_var2, b_var2):
    w1 = jnp.concatenate([w_mean1, w_var1], axis=1)
    b1 = jnp.concatenate([b_mean1, b_var1]).reshape(1, 2 * _HID)
    bm2 = b_mean2.reshape(1, _OUT_F)
    bv2 = b_var2.reshape(1, _OUT_F)

    src3 = edge_index[0].reshape(_NW, _NCHUNK, _K)
    dst3 = edge_index[1].reshape(_NW, _NCHUNK, _K)
    degp = _degree_partials(dst3)
    h1 = _dense1(x, w1, b1)
    t1 = _scale1(h1, degp)
    s1 = _segment_sum_partials(t1, src3, dst3, 2 * _HID, _K)
    t2 = _dense2(s1, w_mean2, w_var2, bm2, bv2, degp)
    s2 = _segment_sum_partials_big(t2, src3, dst3, 2 * _OUT_F)
    eps = jax.random.normal(jax.random.key(42), (_N, _OUT_F), jnp.float32)
    return _final(s2, degp, eps)


# speedup vs baseline: 1.1368x; 1.0301x over previous
import functools

import jax
import jax.numpy as jnp
from jax import lax
from jax.experimental import pallas as pl
from jax.experimental.pallas import tpu as pltpu
from jax.experimental.pallas import tpu_sc as plsc

_N = 10000
_E = 320000
_IN_F = 128
_HID = 16
_OUT_F = 64
_GAMMA = 1.0

_NC = 2
_NS = 16
_NW = _NC * _NS
_EPW = _E // _NW
_K = 80
_NCHUNK = _EPW // _K
_NBUF = 4
_SLAB = _N // _NS
_ZSLAB = 640
_NPAD = _NS * _ZSLAB

_RB = 400
_GRID = _N // _RB

_DEGW = 16


def _degree_partials(dst3):
    mesh = plsc.VectorSubcoreMesh(core_axis_name="c", subcore_axis_name="s")

    @functools.partial(
        pl.kernel,
        mesh=mesh,
        out_type=jax.ShapeDtypeStruct((_NC, _NPAD, _DEGW), jnp.float32),
        scratch_types=[
            pltpu.VMEM((_NCHUNK, _K), jnp.int32),
            pltpu.VMEM((_K, _DEGW), jnp.float32),
            pltpu.VMEM((_K, _DEGW), jnp.float32),
            pltpu.VMEM_SHARED((_NPAD, _DEGW), jnp.float32),
            pltpu.SemaphoreType.DMA,
            pltpu.SemaphoreType.DMA,
        ],
        compiler_params=pltpu.CompilerParams(use_tc_tiling_on_sc=False),
    )
    def deg_kernel(dst_hbm, out, dst_v, ones_v, zero_v, acc, isem, ssem):
        cid = lax.axis_index("c")
        sid = lax.axis_index("s")
        wid = cid * _NS + sid

        idx_load = pltpu.async_copy(dst_hbm.at[wid], dst_v, isem)

        ones_row = jnp.full((16,), 1.0, jnp.float32)
        zero_row = jnp.zeros((16,), jnp.float32)

        def fill(i, _):
            ones_v[i, :] = ones_row
            zero_v[i, :] = zero_row
            return 0

        lax.fori_loop(0, _K, fill, 0)

        zbase = sid * _ZSLAB
        for j in range(_ZSLAB // _K):
            pltpu.sync_copy(zero_v, acc.at[pl.ds(zbase + j * _K, _K)])
        idx_load.wait()
        plsc.subcore_barrier()

        def fire(i, _):
            pltpu.async_copy(ones_v, acc.at[dst_v.at[i]], ssem, add=True)
            return 0

        lax.fori_loop(0, _NCHUNK, fire, 0)

        def drain(i, _):
            pltpu.make_async_copy(ones_v, acc.at[pl.ds(0, _K)], ssem).wait()
            return 0

        lax.fori_loop(0, _NCHUNK, drain, 0)
        plsc.subcore_barrier()

        rbase = sid * _ZSLAB
        pltpu.sync_copy(acc.at[pl.ds(rbase, _ZSLAB)],
                        out.at[cid, pl.ds(rbase, _ZSLAB)])

    return deg_kernel(dst3)


def _segment_sum_partials(table, src3, dst3, feat, k):
    nchunk = _EPW // k
    mesh = plsc.VectorSubcoreMesh(core_axis_name="c", subcore_axis_name="s")

    @functools.partial(
        pl.kernel,
        mesh=mesh,
        out_type=jax.ShapeDtypeStruct((_NC, _NPAD, feat), jnp.float32),
        scratch_types=[
            pltpu.VMEM((nchunk, k), jnp.int32),
            pltpu.VMEM((nchunk, k), jnp.int32),
            pltpu.VMEM((_NBUF, k, feat), jnp.float32),
            pltpu.SemaphoreType.DMA,
            pltpu.SemaphoreType.DMA((_NBUF,)),
            pltpu.SemaphoreType.DMA((_NBUF,)),
            pltpu.VMEM_SHARED((_NPAD, feat), jnp.float32),
        ],
        compiler_params=pltpu.CompilerParams(use_tc_tiling_on_sc=False),
    )
    def seg_kernel(tab, src_hbm, dst_hbm, out, src_v, dst_v, rows, isem,
                   gsem, ssem, acc):
        cid = lax.axis_index("c")
        sid = lax.axis_index("s")
        wid = cid * _NS + sid

        il0 = pltpu.async_copy(src_hbm.at[wid], src_v, isem)
        il1 = pltpu.async_copy(dst_hbm.at[wid], dst_v, isem)

        zero_row = jnp.zeros((16,), jnp.float32)

        def fill(i, _):
            for j in range(feat // 16):
                rows[0, i, pl.ds(j * 16, 16)] = zero_row
            return 0

        lax.fori_loop(0, k, fill, 0)

        zbase = sid * _ZSLAB
        for j in range(_ZSLAB // k):
            pltpu.sync_copy(rows.at[0], acc.at[pl.ds(zbase + j * k, k)])
        il0.wait()
        il1.wait()
        plsc.subcore_barrier()

        def gather(c, b):
            pltpu.async_copy(tab.at[src_v.at[c]], rows.at[b], gsem.at[b])

        def gwait(b):
            pltpu.make_async_copy(tab.at[src_v.at[0]], rows.at[b],
                                  gsem.at[b]).wait()

        def scatter(c, b):
            pltpu.async_copy(rows.at[b], acc.at[dst_v.at[c]], ssem.at[b],
                             add=True)

        def swait(b):
            pltpu.make_async_copy(rows.at[b], acc.at[pl.ds(0, k)],
                                  ssem.at[b]).wait()

        gather(0, 0)
        gather(1, 1)
        for c in range(2):
            gwait(c)
            scatter(c, c)
            gather(c + 2, c + 2)

        def step(c, _):
            b = lax.rem(c, _NBUF)
            bnn = lax.rem(c + 2, _NBUF)
            gwait(b)
            scatter(c, b)
            swait(bnn)
            gather(c + 2, bnn)
            return 0

        lax.fori_loop(2, nchunk - 2, step, 0)
        for c in range(nchunk - 2, nchunk):
            b = c % _NBUF
            gwait(b)
            scatter(c, b)
        for b in range(_NBUF):
            swait(b)
        plsc.subcore_barrier()

        rbase = sid * _ZSLAB
        pltpu.sync_copy(acc.at[pl.ds(rbase, _ZSLAB)],
                        out.at[cid, pl.ds(rbase, _ZSLAB)])

    return seg_kernel(table, src3, dst3)


def _segment_sum_partials_big(table, src3, dst3, feat):
    _NB = 3
    _ND = 8
    mesh = plsc.VectorSubcoreMesh(core_axis_name="c", subcore_axis_name="s")

    @functools.partial(
        pl.kernel,
        mesh=mesh,
        out_type=jax.ShapeDtypeStruct((_NC, _NPAD, feat), jnp.float32),
        scratch_types=[
            pltpu.VMEM((_NCHUNK, _K), jnp.int32),
            pltpu.VMEM((_ND, _K), jnp.int32),
            pltpu.VMEM((_NB, _K, feat), jnp.float32),
            pltpu.SemaphoreType.DMA,
            pltpu.SemaphoreType.DMA((_ND,)),
            pltpu.SemaphoreType.DMA((_NB,)),
            pltpu.SemaphoreType.DMA((_NB,)),
            pltpu.VMEM_SHARED((_NPAD, feat), jnp.float32),
        ],
        compiler_params=pltpu.CompilerParams(use_tc_tiling_on_sc=False),
    )
    def seg_kernel(tab, src_hbm, dst_hbm, out, src_v, dring, rows, isem,
                   dsem, gsem, ssem, acc):
        cid = lax.axis_index("c")
        sid = lax.axis_index("s")
        wid = cid * _NS + sid

        def dload(c):
            s = lax.rem(c, _ND) if not isinstance(c, int) else c % _ND
            pltpu.async_copy(dst_hbm.at[wid, c], dring.at[s], dsem.at[s])

        def iwait(c):
            s = lax.rem(c, _ND) if not isinstance(c, int) else c % _ND
            pltpu.make_async_copy(dst_hbm.at[wid, 0], dring.at[s],
                                  dsem.at[s]).wait()

        il0 = pltpu.async_copy(src_hbm.at[wid], src_v, isem)
        for c in range(_ND - 2):
            dload(c)

        zero_row = jnp.zeros((16,), jnp.float32)

        def fill(i, _):
            for j in range(feat // 16):
                rows[0, i, pl.ds(j * 16, 16)] = zero_row
            return 0

        lax.fori_loop(0, _K, fill, 0)

        zbase = sid * _ZSLAB
        for j in range(_ZSLAB // _K):
            pltpu.sync_copy(rows.at[0], acc.at[pl.ds(zbase + j * _K, _K)])
        il0.wait()
        plsc.subcore_barrier()

        def gather(c, b):
            pltpu.async_copy(tab.at[src_v.at[c]], rows.at[b], gsem.at[b])

        def gwait(b):
            pltpu.make_async_copy(tab.at[src_v.at[0]], rows.at[b],
                                  gsem.at[b]).wait()

        def scatter(c, b):
            s = lax.rem(c, _ND) if not isinstance(c, int) else c % _ND
            pltpu.async_copy(rows.at[b], acc.at[dring.at[s]], ssem.at[b],
                             add=True)

        def swait(b):
            pltpu.make_async_copy(rows.at[b], acc.at[pl.ds(0, _K)],
                                  ssem.at[b]).wait()

        gather(0, 0)
        gather(1, 1)
        gwait(0)
        iwait(0)
        scatter(0, 0)
        gather(2, 2)
        dload(_ND - 2)
        gwait(1)
        iwait(1)
        scatter(1, 1)
        swait(0)
        gather(3, 0)
        dload(_ND - 1)

        def step(c, _):
            b = lax.rem(c, _NB)
            b2 = lax.rem(c + 2, _NB)
            gwait(b)
            iwait(c)
            scatter(c, b)
            swait(b2)
            gather(c + 2, b2)

            @pl.when(c + _ND - 2 < _NCHUNK)
            def _():
                dload(c + _ND - 2)

            return 0

        lax.fori_loop(2, _NCHUNK - 2, step, 0)
        for c in range(_NCHUNK - 2, _NCHUNK):
            b = c % _NB
            gwait(b)
            iwait(c)
            scatter(c, b)
        for b in range(_NB):
            swait(b)
        plsc.subcore_barrier()

        rbase = sid * _ZSLAB
        pltpu.sync_copy(acc.at[pl.ds(rbase, _ZSLAB)],
                        out.at[cid, pl.ds(rbase, _ZSLAB)])

    return seg_kernel(table, src3, dst3)


def _norms(d_ref):
    deg = jnp.maximum(d_ref[0][:, 0:1] + d_ref[1][:, 0:1], 1.0)
    n1 = lax.rsqrt(deg)
    n2 = 1.0 / deg
    return n1, n2


def _dense1_body(x_ref, w_ref, b_ref, d_ref, o_ref):
    n1, n2 = _norms(d_ref)
    h = jnp.dot(x_ref[...], w_ref[...], preferred_element_type=jnp.float32)
    h = jnp.maximum(h + b_ref[...], 0.0)
    hm = h[:, :_HID]
    hv = h[:, _HID:]
    att = jnp.exp(-_GAMMA * hv)
    o_ref[...] = jnp.concatenate([hm * att * n1, hv * att * att * n2], axis=1)


def _dense2_body(s_ref, wm_ref, wv_ref, bm_ref, bv_ref, d_ref, o_ref):
    n1, n2 = _norms(d_ref)
    s = s_ref[0] + s_ref[1]
    mean_in = s[:, :_HID] * n1
    var_in = s[:, _HID:] * n2
    hm = jnp.dot(mean_in, wm_ref[...],
                 preferred_element_type=jnp.float32) + bm_ref[...]
    hv = jnp.dot(var_in, wv_ref[...],
                 preferred_element_type=jnp.float32) + bv_ref[...]
    hv = jnp.maximum(hv, 0.0)
    att = jnp.exp(-_GAMMA * hv)
    o_ref[...] = jnp.concatenate([hm * att * n1, hv * att * att * n2], axis=1)


def _final_body(s_ref, d_ref, e_ref, o_ref):
    n1, n2 = _norms(d_ref)
    s = s_ref[0] + s_ref[1]
    mean = s[:, :_OUT_F] * n1
    var = s[:, _OUT_F:] * n2
    o_ref[...] = e_ref[...] * jnp.sqrt(var + 1e-8) + mean


def _dense1(x, w1, b1, degp):
    return pl.pallas_call(
        _dense1_body,
        grid=(_GRID,),
        in_specs=[
            pl.BlockSpec((_RB, _IN_F), lambda i: (i, 0)),
            pl.BlockSpec((_IN_F, 2 * _HID), lambda i: (0, 0)),
            pl.BlockSpec((1, 2 * _HID), lambda i: (0, 0)),
            pl.BlockSpec((2, _RB, _DEGW), lambda i: (0, i, 0)),
        ],
        out_specs=pl.BlockSpec((_RB, 2 * _HID), lambda i: (i, 0)),
        out_shape=jax.ShapeDtypeStruct((_N, 2 * _HID), jnp.float32),
    )(x, w1, b1, degp)


def _dense2(s1, wm2, wv2, bm2, bv2, degp):
    return pl.pallas_call(
        _dense2_body,
        grid=(_GRID,),
        in_specs=[
            pl.BlockSpec((2, _RB, 2 * _HID), lambda i: (0, i, 0)),
            pl.BlockSpec((_HID, _OUT_F), lambda i: (0, 0)),
            pl.BlockSpec((_HID, _OUT_F), lambda i: (0, 0)),
            pl.BlockSpec((1, _OUT_F), lambda i: (0, 0)),
            pl.BlockSpec((1, _OUT_F), lambda i: (0, 0)),
            pl.BlockSpec((2, _RB, _DEGW), lambda i: (0, i, 0)),
        ],
        out_specs=pl.BlockSpec((_RB, 2 * _OUT_F), lambda i: (i, 0)),
        out_shape=jax.ShapeDtypeStruct((_N, 2 * _OUT_F), jnp.float32),
    )(s1, wm2, wv2, bm2, bv2, degp)


def _final(s2, degp, eps):
    return pl.pallas_call(
        _final_body,
        grid=(_GRID,),
        in_specs=[
            pl.BlockSpec((2, _RB, 2 * _OUT_F), lambda i: (0, i, 0)),
            pl.BlockSpec((2, _RB, _DEGW), lambda i: (0, i, 0)),
            pl.BlockSpec((_RB, _OUT_F), lambda i: (i, 0)),
        ],
        out_specs=pl.BlockSpec((_RB, _OUT_F), lambda i: (i, 0)),
        out_shape=jax.ShapeDtypeStruct((_N, _OUT_F), jnp.float32),
    )(s2, degp, eps)


def kernel(x, edge_index, w_mean1, b_mean1, w_var1, b_var1,
           w_mean2, b_mean2, w_var2, b_var2):
    w1 = jnp.concatenate([w_mean1, w_var1], axis=1)
    b1 = jnp.concatenate([b_mean1, b_var1]).reshape(1, 2 * _HID)
    bm2 = b_mean2.reshape(1, _OUT_F)
    bv2 = b_var2.reshape(1, _OUT_F)

    src3 = edge_index[0].reshape(_NW, _NCHUNK, _K)
    dst3 = edge_index[1].reshape(_NW, _NCHUNK, _K)
    degp = _degree_partials(dst3)
    t1 = _dense1(x, w1, b1, degp)
    s1 = _segment_sum_partials(t1, src3, dst3, 2 * _HID, _K)
    t2 = _dense2(s1, w_mean2, w_var2, bm2, bv2, degp)
    s2 = _segment_sum_partials_big(t2, src3, dst3, 2 * _OUT_F)
    eps = jax.random.normal(jax.random.key(42), (_N, _OUT_F), jnp.float32)
    return _final(s2, degp, eps)


# speedup vs baseline: 1.2335x; 1.0851x over previous
import functools

import jax
import jax.numpy as jnp
from jax import lax
from jax.experimental import pallas as pl
from jax.experimental.pallas import tpu as pltpu
from jax.experimental.pallas import tpu_sc as plsc

_N = 10000
_E = 320000
_IN_F = 128
_HID = 16
_OUT_F = 64
_GAMMA = 1.0

_NC = 2
_NS = 16
_NW = _NC * _NS
_EPW = _E // _NW
_K = 80
_NCHUNK = _EPW // _K
_NBUF = 4
_Q = 5
_NSTEP = _NCHUNK // _Q
_SLAB = _N // _NS
_ZSLAB = 640
_NPAD = _NS * _ZSLAB

_RB = 400
_GRID = _N // _RB

_DEGW = 16


def _degree_partials(dst3):
    mesh = plsc.VectorSubcoreMesh(core_axis_name="c", subcore_axis_name="s")

    @functools.partial(
        pl.kernel,
        mesh=mesh,
        out_type=jax.ShapeDtypeStruct((_NC, _NPAD, _DEGW), jnp.float32),
        scratch_types=[
            pltpu.VMEM((_NSTEP, _Q * _K), jnp.int32),
            pltpu.VMEM((_Q * _K, _DEGW), jnp.float32),
            pltpu.VMEM((_K, _DEGW), jnp.float32),
            pltpu.VMEM_SHARED((_NPAD, _DEGW), jnp.float32),
            pltpu.SemaphoreType.DMA,
            pltpu.SemaphoreType.DMA,
        ],
        compiler_params=pltpu.CompilerParams(use_tc_tiling_on_sc=False),
    )
    def deg_kernel(dst_hbm, out, dst_v, ones_v, zero_v, acc, isem, ssem):
        cid = lax.axis_index("c")
        sid = lax.axis_index("s")
        wid = cid * _NS + sid

        idx_load = pltpu.async_copy(dst_hbm.at[wid], dst_v, isem)

        ones_row = jnp.full((16,), 1.0, jnp.float32)
        zero_row = jnp.zeros((16,), jnp.float32)

        def fill(i, _):
            for q in range(_Q):
                ones_v[q * _K + i, :] = ones_row
            zero_v[i, :] = zero_row
            return 0

        lax.fori_loop(0, _K, fill, 0)

        zbase = sid * _ZSLAB
        for j in range(_ZSLAB // _K):
            pltpu.sync_copy(zero_v, acc.at[pl.ds(zbase + j * _K, _K)])
        idx_load.wait()
        plsc.subcore_barrier()

        def fire(i, _):
            pltpu.async_copy(ones_v, acc.at[dst_v.at[i]], ssem, add=True)
            return 0

        lax.fori_loop(0, _NSTEP, fire, 0)

        def drain(i, _):
            pltpu.make_async_copy(ones_v, acc.at[dst_v.at[0]], ssem).wait()
            return 0

        lax.fori_loop(0, _NSTEP, drain, 0)
        plsc.subcore_barrier()

        rbase = sid * _ZSLAB
        pltpu.sync_copy(acc.at[pl.ds(rbase, _ZSLAB)],
                        out.at[cid, pl.ds(rbase, _ZSLAB)])

    return deg_kernel(dst3)


def _segment_sum_partials(table, src4, dst4, feat):
    _NB = 3
    mesh = plsc.VectorSubcoreMesh(core_axis_name="c", subcore_axis_name="s")

    @functools.partial(
        pl.kernel,
        mesh=mesh,
        out_type=jax.ShapeDtypeStruct((_NC, _NPAD, feat), jnp.float32),
        scratch_types=[
            pltpu.VMEM((_NSTEP, _Q * _K), jnp.int32),
            pltpu.VMEM((_NSTEP, _Q * _K), jnp.int32),
            pltpu.VMEM((_NB, _Q * _K, feat), jnp.float32),
            pltpu.VMEM((_K, feat), jnp.float32),
            pltpu.SemaphoreType.DMA,
            pltpu.SemaphoreType.DMA((_NB,)),
            pltpu.SemaphoreType.DMA((_NB,)),
            pltpu.VMEM_SHARED((_NPAD, feat), jnp.float32),
        ],
        compiler_params=pltpu.CompilerParams(use_tc_tiling_on_sc=False),
    )
    def seg_kernel(tab, src_hbm, dst_hbm, out, src_v, dst_v, rows, zero_v,
                   isem, gsem, ssem, acc):
        cid = lax.axis_index("c")
        sid = lax.axis_index("s")
        wid = cid * _NS + sid

        il0 = pltpu.async_copy(src_hbm.at[wid], src_v, isem)
        il1 = pltpu.async_copy(dst_hbm.at[wid], dst_v, isem)

        zero_row = jnp.zeros((16,), jnp.float32)

        def fill(i, _):
            for j in range(feat // 16):
                zero_v[i, pl.ds(j * 16, 16)] = zero_row
            return 0

        lax.fori_loop(0, _K, fill, 0)

        zbase = sid * _ZSLAB
        for j in range(_ZSLAB // _K):
            pltpu.sync_copy(zero_v, acc.at[pl.ds(zbase + j * _K, _K)])
        il0.wait()
        il1.wait()
        plsc.subcore_barrier()

        def gather(t, b):
            pltpu.async_copy(tab.at[src_v.at[t]], rows.at[b], gsem.at[b])

        def gwait(b):
            pltpu.make_async_copy(tab.at[src_v.at[0]], rows.at[b],
                                  gsem.at[b]).wait()

        def scatter(t, b):
            pltpu.async_copy(rows.at[b], acc.at[dst_v.at[t]], ssem.at[b],
                             add=True)

        def swait(b):
            pltpu.make_async_copy(rows.at[b], acc.at[dst_v.at[0]],
                                  ssem.at[b]).wait()

        gather(0, 0)
        gather(1, 1)
        gwait(0)
        scatter(0, 0)
        gather(2, 2)
        gwait(1)
        scatter(1, 1)
        swait(0)
        gather(3, 0)

        def step(t, _):
            b = lax.rem(t, _NB)
            b2 = lax.rem(t + 2, _NB)
            gwait(b)
            scatter(t, b)
            swait(b2)
            gather(t + 2, b2)
            return 0

        lax.fori_loop(2, _NSTEP - 2, step, 0)
        for t in range(_NSTEP - 2, _NSTEP):
            b = t % _NB
            gwait(b)
            scatter(t, b)
        for b in range(_NB):
            swait(b)
        plsc.subcore_barrier()

        rbase = sid * _ZSLAB
        pltpu.sync_copy(acc.at[pl.ds(rbase, _ZSLAB)],
                        out.at[cid, pl.ds(rbase, _ZSLAB)])

    return seg_kernel(table, src4, dst4)


def _segment_sum_partials_big(table, src3, dst3, feat):
    _NB = 3
    _ND = 8
    mesh = plsc.VectorSubcoreMesh(core_axis_name="c", subcore_axis_name="s")

    @functools.partial(
        pl.kernel,
        mesh=mesh,
        out_type=jax.ShapeDtypeStruct((_NC, _NPAD, feat), jnp.float32),
        scratch_types=[
            pltpu.VMEM((_NCHUNK, _K), jnp.int32),
            pltpu.VMEM((_ND, _K), jnp.int32),
            pltpu.VMEM((_NB, _K, feat), jnp.float32),
            pltpu.SemaphoreType.DMA,
            pltpu.SemaphoreType.DMA((_ND,)),
            pltpu.SemaphoreType.DMA((_NB,)),
            pltpu.SemaphoreType.DMA((_NB,)),
            pltpu.VMEM_SHARED((_NPAD, feat), jnp.float32),
        ],
        compiler_params=pltpu.CompilerParams(use_tc_tiling_on_sc=False),
    )
    def seg_kernel(tab, src_hbm, dst_hbm, out, src_v, dring, rows, isem,
                   dsem, gsem, ssem, acc):
        cid = lax.axis_index("c")
        sid = lax.axis_index("s")
        wid = cid * _NS + sid

        def dload(c):
            s = lax.rem(c, _ND) if not isinstance(c, int) else c % _ND
            pltpu.async_copy(dst_hbm.at[wid, c], dring.at[s], dsem.at[s])

        def iwait(c):
            s = lax.rem(c, _ND) if not isinstance(c, int) else c % _ND
            pltpu.make_async_copy(dst_hbm.at[wid, 0], dring.at[s],
                                  dsem.at[s]).wait()

        il0 = pltpu.async_copy(src_hbm.at[wid], src_v, isem)
        for c in range(_ND - 2):
            dload(c)

        zero_row = jnp.zeros((16,), jnp.float32)

        def fill(i, _):
            for j in range(feat // 16):
                rows[0, i, pl.ds(j * 16, 16)] = zero_row
            return 0

        lax.fori_loop(0, _K, fill, 0)

        zbase = sid * _ZSLAB
        for j in range(_ZSLAB // _K):
            pltpu.sync_copy(rows.at[0], acc.at[pl.ds(zbase + j * _K, _K)])
        il0.wait()
        plsc.subcore_barrier()

        def gather(c, b):
            pltpu.async_copy(tab.at[src_v.at[c]], rows.at[b], gsem.at[b])

        def gwait(b):
            pltpu.make_async_copy(tab.at[src_v.at[0]], rows.at[b],
                                  gsem.at[b]).wait()

        def scatter(c, b):
            s = lax.rem(c, _ND) if not isinstance(c, int) else c % _ND
            pltpu.async_copy(rows.at[b], acc.at[dring.at[s]], ssem.at[b],
                             add=True)

        def swait(b):
            pltpu.make_async_copy(rows.at[b], acc.at[pl.ds(0, _K)],
                                  ssem.at[b]).wait()

        gather(0, 0)
        gather(1, 1)
        gwait(0)
        iwait(0)
        scatter(0, 0)
        gather(2, 2)
        dload(_ND - 2)
        gwait(1)
        iwait(1)
        scatter(1, 1)
        swait(0)
        gather(3, 0)
        dload(_ND - 1)

        def step(c, _):
            b = lax.rem(c, _NB)
            b2 = lax.rem(c + 2, _NB)
            gwait(b)
            iwait(c)
            scatter(c, b)
            swait(b2)
            gather(c + 2, b2)

            @pl.when(c + _ND - 2 < _NCHUNK)
            def _():
                dload(c + _ND - 2)

            return 0

        lax.fori_loop(2, _NCHUNK - 2, step, 0)
        for c in range(_NCHUNK - 2, _NCHUNK):
            b = c % _NB
            gwait(b)
            iwait(c)
            scatter(c, b)
        for b in range(_NB):
            swait(b)
        plsc.subcore_barrier()

        rbase = sid * _ZSLAB
        pltpu.sync_copy(acc.at[pl.ds(rbase, _ZSLAB)],
                        out.at[cid, pl.ds(rbase, _ZSLAB)])

    return seg_kernel(table, src3, dst3)


def _norms(d_ref):
    deg = jnp.maximum(d_ref[0][:, 0:1] + d_ref[1][:, 0:1], 1.0)
    n1 = lax.rsqrt(deg)
    n2 = 1.0 / deg
    return n1, n2


def _dense1_body(x_ref, w_ref, b_ref, d_ref, o_ref):
    n1, n2 = _norms(d_ref)
    h = jnp.dot(x_ref[...], w_ref[...], preferred_element_type=jnp.float32)
    h = jnp.maximum(h + b_ref[...], 0.0)
    hm = h[:, :_HID]
    hv = h[:, _HID:]
    att = jnp.exp(-_GAMMA * hv)
    o_ref[...] = jnp.concatenate([hm * att * n1, hv * att * att * n2], axis=1)


def _dense2_body(s_ref, wm_ref, wv_ref, bm_ref, bv_ref, d_ref, o_ref):
    n1, n2 = _norms(d_ref)
    s = s_ref[0] + s_ref[1]
    mean_in = s[:, :_HID] * n1
    var_in = s[:, _HID:] * n2
    hm = jnp.dot(mean_in, wm_ref[...],
                 preferred_element_type=jnp.float32) + bm_ref[...]
    hv = jnp.dot(var_in, wv_ref[...],
                 preferred_element_type=jnp.float32) + bv_ref[...]
    hv = jnp.maximum(hv, 0.0)
    att = jnp.exp(-_GAMMA * hv)
    o_ref[...] = jnp.concatenate([hm * att * n1, hv * att * att * n2], axis=1)


def _final_body(s_ref, d_ref, e_ref, o_ref):
    n1, n2 = _norms(d_ref)
    s = s_ref[0] + s_ref[1]
    mean = s[:, :_OUT_F] * n1
    var = s[:, _OUT_F:] * n2
    o_ref[...] = e_ref[...] * jnp.sqrt(var + 1e-8) + mean


def _dense1(x, w1, b1, degp):
    return pl.pallas_call(
        _dense1_body,
        grid=(_GRID,),
        in_specs=[
            pl.BlockSpec((_RB, _IN_F), lambda i: (i, 0)),
            pl.BlockSpec((_IN_F, 2 * _HID), lambda i: (0, 0)),
            pl.BlockSpec((1, 2 * _HID), lambda i: (0, 0)),
            pl.BlockSpec((2, _RB, _DEGW), lambda i: (0, i, 0)),
        ],
        out_specs=pl.BlockSpec((_RB, 2 * _HID), lambda i: (i, 0)),
        out_shape=jax.ShapeDtypeStruct((_N, 2 * _HID), jnp.float32),
    )(x, w1, b1, degp)


def _dense2(s1, wm2, wv2, bm2, bv2, degp):
    return pl.pallas_call(
        _dense2_body,
        grid=(_GRID,),
        in_specs=[
            pl.BlockSpec((2, _RB, 2 * _HID), lambda i: (0, i, 0)),
            pl.BlockSpec((_HID, _OUT_F), lambda i: (0, 0)),
            pl.BlockSpec((_HID, _OUT_F), lambda i: (0, 0)),
            pl.BlockSpec((1, _OUT_F), lambda i: (0, 0)),
            pl.BlockSpec((1, _OUT_F), lambda i: (0, 0)),
            pl.BlockSpec((2, _RB, _DEGW), lambda i: (0, i, 0)),
        ],
        out_specs=pl.BlockSpec((_RB, 2 * _OUT_F), lambda i: (i, 0)),
        out_shape=jax.ShapeDtypeStruct((_N, 2 * _OUT_F), jnp.float32),
    )(s1, wm2, wv2, bm2, bv2, degp)


def _final(s2, degp, eps):
    return pl.pallas_call(
        _final_body,
        grid=(_GRID,),
        in_specs=[
            pl.BlockSpec((2, _RB, 2 * _OUT_F), lambda i: (0, i, 0)),
            pl.BlockSpec((2, _RB, _DEGW), lambda i: (0, i, 0)),
            pl.BlockSpec((_RB, _OUT_F), lambda i: (i, 0)),
        ],
        out_specs=pl.BlockSpec((_RB, _OUT_F), lambda i: (i, 0)),
        out_shape=jax.ShapeDtypeStruct((_N, _OUT_F), jnp.float32),
    )(s2, degp, eps)


def kernel(x, edge_index, w_mean1, b_mean1, w_var1, b_var1,
           w_mean2, b_mean2, w_var2, b_var2):
    w1 = jnp.concatenate([w_mean1, w_var1], axis=1)
    b1 = jnp.concatenate([b_mean1, b_var1]).reshape(1, 2 * _HID)
    bm2 = b_mean2.reshape(1, _OUT_F)
    bv2 = b_var2.reshape(1, _OUT_F)

    src3 = edge_index[0].reshape(_NW, _NCHUNK, _K)
    dst3 = edge_index[1].reshape(_NW, _NCHUNK, _K)
    src4 = edge_index[0].reshape(_NW, _NSTEP, _Q * _K)
    dst4 = edge_index[1].reshape(_NW, _NSTEP, _Q * _K)
    degp = _degree_partials(dst4)
    t1 = _dense1(x, w1, b1, degp)
    s1 = _segment_sum_partials(t1, src4, dst4, 2 * _HID)
    t2 = _dense2(s1, w_mean2, w_var2, bm2, bv2, degp)
    s2 = _segment_sum_partials_big(t2, src3, dst3, 2 * _OUT_F)
    eps = jax.random.normal(jax.random.key(42), (_N, _OUT_F), jnp.float32)
    return _final(s2, degp, eps)
